# Initial kernel scaffold; baseline (speedup 1.0000x reference)
import jax
import jax.numpy as jnp
from jax.experimental import pallas as pl


def kernel(x, table):
    raise NotImplementedError("write your pallas kernel here")


# speedup vs baseline: 1.2297x; 1.2297x over previous
import functools

import jax
import jax.numpy as jnp
from jax import lax
from jax.experimental import pallas as pl
from jax.experimental.pallas import tpu as pltpu
from jax.experimental.pallas import tpu_sc as plsc

_NUM_CORES = 2
_NUM_SUBCORES = 16
_NUM_WORKERS = _NUM_CORES * _NUM_SUBCORES


@functools.lru_cache(maxsize=None)
def _make_gather(V, D, N):
    n_per_w = N // _NUM_WORKERS
    chunk = 80
    n_chunks = n_per_w // chunk
    assert n_per_w % chunk == 0 and chunk % 8 == 0

    mesh = plsc.VectorSubcoreMesh(
        core_axis_name="c", subcore_axis_name="s",
        num_cores=_NUM_CORES, num_subcores=_NUM_SUBCORES)

    @functools.partial(
        pl.kernel,
        out_type=jax.ShapeDtypeStruct((N, D), jnp.float32),
        mesh=mesh,
        scratch_types=[
            pltpu.VMEM((n_per_w,), jnp.int32),
            pltpu.VMEM((chunk, D), jnp.float32),
            pltpu.SemaphoreType.DMA,
        ],
    )
    def gather_kernel(idx_hbm, table_hbm, out_hbm, idx_v, rows_v, sem):
        wid = lax.axis_index("s") * _NUM_CORES + lax.axis_index("c")
        base = wid * n_per_w
        pltpu.sync_copy(idx_hbm.at[pl.ds(base, n_per_w)], idx_v)

        def body(c, carry):
            off = c * chunk
            pltpu.async_copy(
                table_hbm.at[idx_v.at[pl.ds(off, chunk)]], rows_v, sem
            ).wait()
            pltpu.sync_copy(rows_v, out_hbm.at[pl.ds(base + off, chunk)])
            return carry

        lax.fori_loop(0, n_chunks, body, 0)

    return gather_kernel


def kernel(x, table):
    B, L = x.shape
    V, D = table.shape
    N = B * L
    xf = x.reshape(N).astype(jnp.int32)
    out = _make_gather(V, D, N)(xf, table)
    return out.reshape(B, L, D)


# speedup vs baseline: 1.2802x; 1.0411x over previous
import functools

import jax
import jax.numpy as jnp
from jax import lax
from jax.experimental import pallas as pl
from jax.experimental.pallas import tpu as pltpu
from jax.experimental.pallas import tpu_sc as plsc

_NUM_CORES = 2
_NUM_SUBCORES = 16
_NUM_WORKERS = _NUM_CORES * _NUM_SUBCORES


@functools.lru_cache(maxsize=None)
def _make_gather(V, D, N, chunk=80, nbuf=2):
    n_per_w = N // _NUM_WORKERS
    n_chunks = n_per_w // chunk
    n_groups = n_chunks // nbuf
    assert n_per_w % chunk == 0 and chunk % 8 == 0 and n_chunks % nbuf == 0

    mesh = plsc.VectorSubcoreMesh(
        core_axis_name="c", subcore_axis_name="s",
        num_cores=_NUM_CORES, num_subcores=_NUM_SUBCORES)

    @functools.partial(
        pl.kernel,
        out_type=jax.ShapeDtypeStruct((N, D), jnp.float32),
        mesh=mesh,
        scratch_types=[
            pltpu.VMEM((n_per_w,), jnp.int32),
            [pltpu.VMEM((chunk, D), jnp.float32) for _ in range(nbuf)],
            [pltpu.SemaphoreType.DMA for _ in range(nbuf)],
            [pltpu.SemaphoreType.DMA for _ in range(nbuf)],
        ],
    )
    def gather_kernel(idx_hbm, table_hbm, out_hbm, idx_v, rows, gsem, wsem):
        wid = lax.axis_index("s") * _NUM_CORES + lax.axis_index("c")
        base = wid * n_per_w
        pltpu.sync_copy(idx_hbm.at[pl.ds(base, n_per_w)], idx_v)

        def fire_gather(c, b):
            pltpu.async_copy(
                table_hbm.at[idx_v.at[pl.ds(c * chunk, chunk)]],
                rows[b], gsem[b])

        def wait_gather(c, b):
            pltpu.make_async_copy(
                table_hbm.at[idx_v.at[pl.ds(c * chunk, chunk)]],
                rows[b], gsem[b]).wait()

        def fire_write(c, b):
            pltpu.async_copy(
                rows[b], out_hbm.at[pl.ds(base + c * chunk, chunk)], wsem[b])

        def wait_write(c, b):
            pltpu.make_async_copy(
                rows[b], out_hbm.at[pl.ds(base + c * chunk, chunk)],
                wsem[b]).wait()

        for b in range(nbuf):
            fire_gather(b, b)

        def body(i, carry):
            cc = i * nbuf
            for b in range(nbuf):
                c = cc + b
                wait_gather(c, b)
                fire_write(c, b)
                wait_write(c, b)
                fire_gather(c + nbuf, b)
            return carry

        lax.fori_loop(0, n_groups - 1, body, 0)

        last = (n_groups - 1) * nbuf
        for b in range(nbuf):
            wait_gather(last + b, b)
            fire_write(last + b, b)
        for b in range(nbuf):
            wait_write(last + b, b)

    return gather_kernel


def kernel(x, table):
    B, L = x.shape
    V, D = table.shape
    N = B * L
    xf = x.reshape(N).astype(jnp.int32)
    out = _make_gather(V, D, N)(xf, table)
    return out.reshape(B, L, D)


# speedup vs baseline: 1.2829x; 1.0021x over previous
import functools

import jax
import jax.numpy as jnp
from jax import lax
from jax.experimental import pallas as pl
from jax.experimental.pallas import tpu as pltpu
from jax.experimental.pallas import tpu_sc as plsc

_NUM_CORES = 2
_NUM_SUBCORES = 16
_NUM_WORKERS = _NUM_CORES * _NUM_SUBCORES


@functools.lru_cache(maxsize=None)
def _make_gather(V, D, N, chunk=40, nbuf=4):
    n_per_w = N // _NUM_WORKERS
    n_chunks = n_per_w // chunk
    n_groups = n_chunks // nbuf
    assert n_per_w % chunk == 0 and chunk % 8 == 0 and n_chunks % nbuf == 0

    mesh = plsc.VectorSubcoreMesh(
        core_axis_name="c", subcore_axis_name="s",
        num_cores=_NUM_CORES, num_subcores=_NUM_SUBCORES)

    @functools.partial(
        pl.kernel,
        out_type=jax.ShapeDtypeStruct((N, D), jnp.float32),
        mesh=mesh,
        scratch_types=[
            pltpu.VMEM((n_per_w,), jnp.int32),
            [pltpu.VMEM((chunk, D), jnp.float32) for _ in range(nbuf)],
            [pltpu.SemaphoreType.DMA for _ in range(nbuf)],
            [pltpu.SemaphoreType.DMA for _ in range(nbuf)],
        ],
    )
    def gather_kernel(idx_hbm, table_hbm, out_hbm, idx_v, rows, gsem, wsem):
        wid = lax.axis_index("s") * _NUM_CORES + lax.axis_index("c")
        base = wid * n_per_w
        pltpu.sync_copy(idx_hbm.at[pl.ds(base, n_per_w)], idx_v)

        def fire_gather(c, b):
            pltpu.async_copy(
                table_hbm.at[idx_v.at[pl.ds(c * chunk, chunk)]],
                rows[b], gsem[b])

        def wait_gather(c, b):
            pltpu.make_async_copy(
                table_hbm.at[idx_v.at[pl.ds(c * chunk, chunk)]],
                rows[b], gsem[b]).wait()

        def fire_write(c, b):
            pltpu.async_copy(
                rows[b], out_hbm.at[pl.ds(base + c * chunk, chunk)], wsem[b])

        def wait_write(c, b):
            pltpu.make_async_copy(
                rows[b], out_hbm.at[pl.ds(base + c * chunk, chunk)],
                wsem[b]).wait()

        for b in range(nbuf):
            fire_gather(b, b)

        def body(i, carry):
            cc = i * nbuf
            for b in range(nbuf):
                c = cc + b
                wait_gather(c, b)
                fire_write(c, b)
                wait_write(c, b)
                fire_gather(c + nbuf, b)
            return carry

        lax.fori_loop(0, n_groups - 1, body, 0)

        last = (n_groups - 1) * nbuf
        for b in range(nbuf):
            wait_gather(last + b, b)
            fire_write(last + b, b)
        for b in range(nbuf):
            wait_write(last + b, b)

    return gather_kernel


def kernel(x, table):
    B, L = x.shape
    V, D = table.shape
    N = B * L
    xf = x.reshape(N).astype(jnp.int32)
    out = _make_gather(V, D, N)(xf, table)
    return out.reshape(B, L, D)


# speedup vs baseline: 1.8175x; 1.4168x over previous
import functools

import jax
import jax.numpy as jnp
from jax import lax
from jax.experimental import pallas as pl
from jax.experimental.pallas import tpu as pltpu
from jax.experimental.pallas import tpu_sc as plsc

_NUM_CORES = 2
_NUM_SUBCORES = 16
_NUM_WORKERS = _NUM_CORES * _NUM_SUBCORES


@functools.lru_cache(maxsize=None)
def _make_gather(V, D, B, L, nbuf=4):
    b_per_w = B // _NUM_WORKERS
    n_groups = b_per_w // nbuf
    assert B % _NUM_WORKERS == 0 and b_per_w % nbuf == 0

    mesh = plsc.VectorSubcoreMesh(
        core_axis_name="c", subcore_axis_name="s",
        num_cores=_NUM_CORES, num_subcores=_NUM_SUBCORES)

    @functools.partial(
        pl.kernel,
        out_type=jax.ShapeDtypeStruct((B, L, D), jnp.float32),
        mesh=mesh,
        scratch_types=[
            pltpu.VMEM((b_per_w, L), jnp.int32),
            [pltpu.VMEM((L, D), jnp.float32) for _ in range(nbuf)],
            [pltpu.SemaphoreType.DMA for _ in range(nbuf)],
            [pltpu.SemaphoreType.DMA for _ in range(nbuf)],
        ],
    )
    def gather_kernel(idx_hbm, table_hbm, out_hbm, idx_v, rows, gsem, wsem):
        wid = lax.axis_index("s") * _NUM_CORES + lax.axis_index("c")
        base = wid * b_per_w
        pltpu.sync_copy(idx_hbm.at[pl.ds(base, b_per_w)], idx_v)

        def fire_gather(c, b):
            pltpu.async_copy(table_hbm.at[idx_v.at[c]], rows[b], gsem[b])

        def wait_gather(c, b):
            pltpu.make_async_copy(
                table_hbm.at[idx_v.at[c]], rows[b], gsem[b]).wait()

        def fire_write(c, b):
            pltpu.async_copy(rows[b], out_hbm.at[base + c], wsem[b])

        def wait_write(c, b):
            pltpu.make_async_copy(
                rows[b], out_hbm.at[base + c], wsem[b]).wait()

        for b in range(nbuf):
            fire_gather(b, b)

        def body(i, carry):
            cc = i * nbuf
            for b in range(nbuf):
                c = cc + b
                wait_gather(c, b)
                fire_write(c, b)
                wait_write(c, b)
                fire_gather(c + nbuf, b)
            return carry

        lax.fori_loop(0, n_groups - 1, body, 0)

        last = (n_groups - 1) * nbuf
        for b in range(nbuf):
            wait_gather(last + b, b)
            fire_write(last + b, b)
        for b in range(nbuf):
            wait_write(last + b, b)

    return gather_kernel


def kernel(x, table):
    B, L = x.shape
    V, D = table.shape
    return _make_gather(V, D, B, L)(x.astype(jnp.int32), table)
